# baseline (speedup 1.0000x reference)
import jax
import jax.numpy as jnp
from jax import lax
from jax.experimental import pallas as pl
from jax.experimental.pallas import tpu as pltpu
from jax.experimental.pallas import tpu_sc as plsc

B, C, H, W = 8, 3, 512, 512
MAX_DISP = 32.0
NC, NS = 2, 16
NW = NC * NS
W_PER_B = NW // B
ROWS_PER_W = H // W_PER_B
RBLK = 16
NBLK = ROWS_PER_W // RBLK
WPAD = W + 1
F2P23 = 8388608.0


def _body(image_hbm, depth_hbm, out_l_hbm, out_r_hbm,
          dep_s, im0_s, im1_s, im2_s, cnt_s, ac0_s, ac1_s, ac2_s, out_s,
          out_sem0, out_sem1):
    wid = lax.axis_index("s") * NC + lax.axis_index("c")
    b = wid // W_PER_B
    y_base = (wid % W_PER_B) * ROWS_PER_W
    im_s = (im0_s, im1_s, im2_s)
    ac_s = (ac0_s, ac1_s, ac2_s)
    out_sems = (out_sem0, out_sem1)
    out_hbms = (out_l_hbm, out_r_hbm)

    lane16 = lax.iota(jnp.int32, 16)
    ones = jnp.ones((16,), jnp.float32)
    zeros = jnp.zeros((16,), jnp.float32)

    def zx(x, c2):
        cnt_s[x, :] = zeros
        for c in range(C):
            ac_s[c][x, :] = zeros
        return c2
    lax.fori_loop(0, W, zx, 0)

    def wait_out(ph):
        pltpu.make_async_copy(
            out_hbms[ph].at[0, :, pl.ds(0, RBLK), :],
            out_s.at[ph, :, :, pl.ds(0, W)],
            out_sems[ph]).wait()

    def do_block(blk, carry):
        ys = y_base + blk * RBLK
        pltpu.sync_copy(depth_hbm.at[b, pl.ds(ys, RBLK), :],
                        dep_s.at[:, pl.ds(0, W)])
        for c in range(C):
            pltpu.sync_copy(image_hbm.at[b, c, pl.ds(ys, RBLK), :],
                            im_s[c].at[:, pl.ds(0, W)])

        for di in range(2):
            @pl.when(blk > 0)
            def _():
                wait_out(di)

            def sx(x, c2):
                xs = jnp.broadcast_to(x, (16,))
                dep = plsc.load_gather(dep_s, [lane16, xs])
                disp = ((dep * MAX_DISP + F2P23) - F2P23).astype(jnp.int32)
                xt = xs - disp if di == 0 else xs + disp
                msk = (xt >= 0) & (xt < W)
                xtc = jnp.clip(xt, 0, W - 1)
                plsc.addupdate_scatter(cnt_s, [xtc, lane16], ones, mask=msk)
                for c in range(C):
                    v = plsc.load_gather(im_s[c], [lane16, xs])
                    plsc.addupdate_scatter(ac_s[c], [xtc, lane16], v, mask=msk)
                return c2
            lax.fori_loop(0, W, sx, 0)

            def fx(x, c2):
                xs = jnp.broadcast_to(x, (16,))
                cnt = cnt_s[x, :]
                inv = 1.0 / jnp.maximum(cnt, 1.0)
                cnt_s[x, :] = zeros
                for c in range(C):
                    a = ac_s[c][x, :]
                    o = jnp.clip(a * inv, 0.0, 1.0)
                    plsc.store_scatter(out_s.at[di, c], [lane16, xs], o)
                    ac_s[c][x, :] = zeros
                return c2
            lax.fori_loop(0, W, fx, 0)

            pltpu.async_copy(out_s.at[di, :, :, pl.ds(0, W)],
                             out_hbms[di].at[b, :, pl.ds(ys, RBLK), :],
                             out_sems[di])
        return carry

    lax.fori_loop(0, NBLK, do_block, 0)
    wait_out(0)
    wait_out(1)


def kernel(image, depth):
    mesh = plsc.VectorSubcoreMesh(core_axis_name="c", subcore_axis_name="s",
                                  num_cores=NC, num_subcores=NS)
    f = pl.kernel(
        _body,
        out_type=(jax.ShapeDtypeStruct((B, C, H, W), jnp.float32),
                  jax.ShapeDtypeStruct((B, C, H, W), jnp.float32)),
        mesh=mesh,
        scratch_types=[
            pltpu.VMEM((RBLK, WPAD), jnp.float32),
            pltpu.VMEM((RBLK, WPAD), jnp.float32),
            pltpu.VMEM((RBLK, WPAD), jnp.float32),
            pltpu.VMEM((RBLK, WPAD), jnp.float32),
            pltpu.VMEM((W, RBLK), jnp.float32),
            pltpu.VMEM((W, RBLK), jnp.float32),
            pltpu.VMEM((W, RBLK), jnp.float32),
            pltpu.VMEM((W, RBLK), jnp.float32),
            pltpu.VMEM((2, C, RBLK, WPAD), jnp.float32),
            pltpu.SemaphoreType.DMA,
            pltpu.SemaphoreType.DMA,
        ],
        compiler_params=pltpu.CompilerParams(use_tc_tiling_on_sc=False,
                                             needs_layout_passes=False),
    )
    return f(image, depth)


# speedup vs baseline: 1.2732x; 1.2732x over previous
import jax
import jax.numpy as jnp
from jax import lax
from jax.experimental import pallas as pl
from jax.experimental.pallas import tpu as pltpu
from jax.experimental.pallas import tpu_sc as plsc

B, C, H, W = 8, 3, 512, 512
MAX_DISP = 32.0
NC, NS = 2, 16
NW = NC * NS
W_PER_B = NW // B
ROWS_PER_W = H // W_PER_B
RBLK = 16
NBLK = ROWS_PER_W // RBLK
NCH = W // 16
F2P23 = 8388608.0


def _body(image_hbm, depth_hbm, out_l_hbm, out_r_hbm,
          depth_v, img_v, cn0, cn1, ac00, ac01, ac02, ac10, ac11, ac12):
    wid = lax.axis_index("s") * NC + lax.axis_index("c")
    b = wid // W_PER_B
    y_base = (wid % W_PER_B) * ROWS_PER_W
    cnt = (cn0, cn1)
    acc = ((ac00, ac01, ac02), (ac10, ac11, ac12))

    xiota = lax.iota(jnp.int32, 16)
    ones = jnp.ones((16,), jnp.float32)
    zeros = jnp.zeros((16,), jnp.float32)

    def do_block(blk, carry):
        ys = y_base + blk * RBLK
        pltpu.sync_copy(depth_hbm.at[b, pl.ds(ys, RBLK), :], depth_v)
        pltpu.sync_copy(image_hbm.at[b, :, pl.ds(ys, RBLK), :], img_v)

        def zrow(r, c1):
            def zj(j, c2):
                xo = j * 16
                for di in range(2):
                    cnt[di][r, pl.ds(xo, 16)] = zeros
                    for c in range(C):
                        acc[di][c][r, pl.ds(xo, 16)] = zeros
                return c2
            lax.fori_loop(0, NCH, zj, 0)
            return c1
        lax.fori_loop(0, RBLK, zrow, 0)

        def srow(r, c1):
            rr = jnp.broadcast_to(r, (16,))

            def sj(j, c2):
                xo = j * 16
                d16 = depth_v[r, pl.ds(xo, 16)]
                disp = ((d16 * MAX_DISP + F2P23) - F2P23).astype(jnp.int32)
                xb = xiota + xo
                vals = [img_v[c, r, pl.ds(xo, 16)] for c in range(C)]
                for di in range(2):
                    xt = xb - disp if di == 0 else xb + disp
                    msk = (xt >= 0) & (xt < W)
                    xtc = jnp.clip(xt, 0, W - 1)
                    plsc.addupdate_scatter(cnt[di], [rr, xtc], ones, mask=msk)
                    for c in range(C):
                        plsc.addupdate_scatter(acc[di][c], [rr, xtc], vals[c],
                                               mask=msk)
                return c2
            lax.fori_loop(0, NCH, sj, 0)
            return c1
        lax.fori_loop(0, RBLK, srow, 0)

        def frow(r, c1):
            def fj(j, c2):
                xo = j * 16
                for di in range(2):
                    cv = cnt[di][r, pl.ds(xo, 16)]
                    inv = 1.0 / jnp.maximum(cv, 1.0)
                    for c in range(C):
                        a = acc[di][c][r, pl.ds(xo, 16)]
                        acc[di][c][r, pl.ds(xo, 16)] = jnp.clip(a * inv,
                                                                0.0, 1.0)
                return c2
            lax.fori_loop(0, NCH, fj, 0)
            return c1
        lax.fori_loop(0, RBLK, frow, 0)

        for di, out_hbm in ((0, out_l_hbm), (1, out_r_hbm)):
            for c in range(C):
                pltpu.sync_copy(acc[di][c],
                                out_hbm.at[b, c, pl.ds(ys, RBLK), :])
        return carry

    lax.fori_loop(0, NBLK, do_block, 0)


def kernel(image, depth):
    mesh = plsc.VectorSubcoreMesh(core_axis_name="c", subcore_axis_name="s",
                                  num_cores=NC, num_subcores=NS)
    f = pl.kernel(
        _body,
        out_type=(jax.ShapeDtypeStruct((B, C, H, W), jnp.float32),
                  jax.ShapeDtypeStruct((B, C, H, W), jnp.float32)),
        mesh=mesh,
        scratch_types=[
            pltpu.VMEM((RBLK, W), jnp.float32),
            pltpu.VMEM((C, RBLK, W), jnp.float32),
            pltpu.VMEM((RBLK, W), jnp.float32),
            pltpu.VMEM((RBLK, W), jnp.float32),
            pltpu.VMEM((RBLK, W), jnp.float32),
            pltpu.VMEM((RBLK, W), jnp.float32),
            pltpu.VMEM((RBLK, W), jnp.float32),
            pltpu.VMEM((RBLK, W), jnp.float32),
            pltpu.VMEM((RBLK, W), jnp.float32),
            pltpu.VMEM((RBLK, W), jnp.float32),
        ],
        compiler_params=pltpu.CompilerParams(use_tc_tiling_on_sc=False,
                                             needs_layout_passes=False),
    )
    return f(image, depth)


# speedup vs baseline: 1.5334x; 1.2043x over previous
import jax
import jax.numpy as jnp
from jax import lax
from jax.experimental import pallas as pl
from jax.experimental.pallas import tpu as pltpu
from jax.experimental.pallas import tpu_sc as plsc

B, C, H, W = 8, 3, 512, 512
MAX_DISP = 32.0
NC, NS = 2, 16
NW = NC * NS
W_PER_B = NW // B
ROWS_PER_W = H // W_PER_B
RBLK = 16
NBLK = ROWS_PER_W // RBLK
NCH = W // 16
F2P23 = 8388608.0


def _body(image_hbm, depth_hbm, out_l_hbm, out_r_hbm,
          depth_v, img_v, cnt_v, acc_v):
    wid = lax.axis_index("s") * NC + lax.axis_index("c")
    b = wid // W_PER_B
    y_base = (wid % W_PER_B) * ROWS_PER_W

    xiota = lax.iota(jnp.int32, 16)
    ones = jnp.ones((16,), jnp.float32)
    zeros = jnp.zeros((16,), jnp.float32)
    dvecs = [jnp.full((16,), di, jnp.int32) for di in range(2)]
    cvecs = [jnp.full((16,), c, jnp.int32) for c in range(C)]

    def do_block(blk, carry):
        ys = y_base + blk * RBLK
        pltpu.sync_copy(depth_hbm.at[b, pl.ds(ys, RBLK), :], depth_v)
        pltpu.sync_copy(image_hbm.at[b, :, pl.ds(ys, RBLK), :], img_v)

        def zero_k(k, c2):
            r = k // NCH
            xo = (k % NCH) * 16
            for di in range(2):
                cnt_v[di, r, pl.ds(xo, 16)] = zeros
                for c in range(C):
                    acc_v[di, c, r, pl.ds(xo, 16)] = zeros
            return c2
        lax.fori_loop(0, RBLK * NCH, zero_k, 0)

        def scat_k(k, c2):
            r = k // NCH
            xo = (k % NCH) * 16
            d16 = depth_v[r, pl.ds(xo, 16)]
            disp = ((d16 * MAX_DISP + F2P23) - F2P23).astype(jnp.int32)
            xb = xiota + xo
            rr = jnp.broadcast_to(r, (16,))
            vals = [img_v[c, r, pl.ds(xo, 16)] for c in range(C)]
            for di in range(2):
                xt = xb - disp if di == 0 else xb + disp
                msk = (xt >= 0) & (xt < W)
                xtc = jnp.clip(xt, 0, W - 1)
                plsc.addupdate_scatter(cnt_v, [dvecs[di], rr, xtc],
                                       ones, mask=msk)
                for c in range(C):
                    plsc.addupdate_scatter(acc_v, [dvecs[di], cvecs[c], rr, xtc],
                                           vals[c], mask=msk)
            return c2
        lax.fori_loop(0, RBLK * NCH, scat_k, 0)

        def fin_k(k, c2):
            r = k // NCH
            xo = (k % NCH) * 16
            for di in range(2):
                cnt = cnt_v[di, r, pl.ds(xo, 16)]
                inv = 1.0 / jnp.maximum(cnt, 1.0)
                for c in range(C):
                    a = acc_v[di, c, r, pl.ds(xo, 16)]
                    acc_v[di, c, r, pl.ds(xo, 16)] = jnp.clip(a * inv, 0.0, 1.0)
            return c2
        lax.fori_loop(0, RBLK * NCH, fin_k, 0)

        pltpu.sync_copy(acc_v.at[0], out_l_hbm.at[b, :, pl.ds(ys, RBLK), :])
        pltpu.sync_copy(acc_v.at[1], out_r_hbm.at[b, :, pl.ds(ys, RBLK), :])
        return carry

    lax.fori_loop(0, NBLK, do_block, 0)


def kernel(image, depth):
    mesh = plsc.VectorSubcoreMesh(core_axis_name="c", subcore_axis_name="s",
                                  num_cores=NC, num_subcores=NS)
    f = pl.kernel(
        _body,
        out_type=(jax.ShapeDtypeStruct((B, C, H, W), jnp.float32),
                  jax.ShapeDtypeStruct((B, C, H, W), jnp.float32)),
        mesh=mesh,
        scratch_types=[
            pltpu.VMEM((RBLK, W), jnp.float32),
            pltpu.VMEM((C, RBLK, W), jnp.float32),
            pltpu.VMEM((2, RBLK, W), jnp.float32),
            pltpu.VMEM((2, C, RBLK, W), jnp.float32),
        ],
        compiler_params=pltpu.CompilerParams(use_tc_tiling_on_sc=False,
                                             needs_layout_passes=False),
    )
    return f(image, depth)


# speedup vs baseline: 2.5117x; 1.6381x over previous
import jax
import jax.numpy as jnp
from jax import lax
from jax.experimental import pallas as pl
from jax.experimental.pallas import tpu as pltpu
from jax.experimental.pallas import tpu_sc as plsc

B, C, H, W = 8, 3, 512, 512
MAX_DISP = 32.0
NC, NS = 2, 16
NW = NC * NS
W_PER_B = NW // B
ROWS_PER_W = H // W_PER_B
RBLK = 16
NBLK = ROWS_PER_W // RBLK
NCH = W // 16
F2P23 = 8388608.0


def _body(image_hbm, depth_hbm, out_l_hbm, out_r_hbm,
          depth_v, img_v, cnt_v, acc_v):
    wid = lax.axis_index("s") * NC + lax.axis_index("c")
    b = wid // W_PER_B
    y_base = (wid % W_PER_B) * ROWS_PER_W

    xiota = lax.iota(jnp.int32, 16)
    ones = jnp.ones((16,), jnp.float32)
    zeros = jnp.zeros((16,), jnp.float32)
    dvecs = [jnp.full((16,), di, jnp.int32) for di in range(2)]
    cvecs = [jnp.full((16,), c, jnp.int32) for c in range(C)]

    def do_block(blk, carry):
        ys = y_base + blk * RBLK
        pltpu.sync_copy(depth_hbm.at[b, pl.ds(ys, RBLK), :], depth_v)
        pltpu.sync_copy(image_hbm.at[b, :, pl.ds(ys, RBLK), :], img_v)

        @plsc.parallel_loop(0, RBLK * NCH, 1, unroll=2)
        def zero_k(k):
            r = k // NCH
            xo = (k % NCH) * 16
            for di in range(2):
                cnt_v[di, r, pl.ds(xo, 16)] = zeros
                for c in range(C):
                    acc_v[di, c, r, pl.ds(xo, 16)] = zeros

        @plsc.parallel_loop(0, RBLK * NCH, 1, unroll=2)
        def scat_k(k):
            r = k // NCH
            xo = (k % NCH) * 16
            d16 = depth_v[r, pl.ds(xo, 16)]
            disp = ((d16 * MAX_DISP + F2P23) - F2P23).astype(jnp.int32)
            xb = xiota + xo
            rr = jnp.broadcast_to(r, (16,))
            vals = [img_v[c, r, pl.ds(xo, 16)] for c in range(C)]
            for di in range(2):
                xt = xb - disp if di == 0 else xb + disp
                msk = (xt >= 0) & (xt < W)
                xtc = jnp.clip(xt, 0, W - 1)
                plsc.addupdate_scatter(cnt_v, [dvecs[di], rr, xtc],
                                       ones, mask=msk)
                for c in range(C):
                    plsc.addupdate_scatter(acc_v, [dvecs[di], cvecs[c], rr, xtc],
                                           vals[c], mask=msk)

        @plsc.parallel_loop(0, RBLK * NCH, 1, unroll=2)
        def fin_k(k):
            r = k // NCH
            xo = (k % NCH) * 16
            for di in range(2):
                cnt = cnt_v[di, r, pl.ds(xo, 16)]
                inv = 1.0 / jnp.maximum(cnt, 1.0)
                for c in range(C):
                    a = acc_v[di, c, r, pl.ds(xo, 16)]
                    acc_v[di, c, r, pl.ds(xo, 16)] = jnp.clip(a * inv, 0.0, 1.0)

        pltpu.sync_copy(acc_v.at[0], out_l_hbm.at[b, :, pl.ds(ys, RBLK), :])
        pltpu.sync_copy(acc_v.at[1], out_r_hbm.at[b, :, pl.ds(ys, RBLK), :])
        return carry

    lax.fori_loop(0, NBLK, do_block, 0)


def kernel(image, depth):
    mesh = plsc.VectorSubcoreMesh(core_axis_name="c", subcore_axis_name="s",
                                  num_cores=NC, num_subcores=NS)
    f = pl.kernel(
        _body,
        out_type=(jax.ShapeDtypeStruct((B, C, H, W), jnp.float32),
                  jax.ShapeDtypeStruct((B, C, H, W), jnp.float32)),
        mesh=mesh,
        scratch_types=[
            pltpu.VMEM((RBLK, W), jnp.float32),
            pltpu.VMEM((C, RBLK, W), jnp.float32),
            pltpu.VMEM((2, RBLK, W), jnp.float32),
            pltpu.VMEM((2, C, RBLK, W), jnp.float32),
        ],
        compiler_params=pltpu.CompilerParams(use_tc_tiling_on_sc=False,
                                             needs_layout_passes=False),
    )
    return f(image, depth)
